# baseline (speedup 1.0000x reference)
import functools

import jax
import jax.numpy as jnp
from jax.experimental import pallas as pl
from jax.experimental.pallas import tpu as pltpu


def _se_kernel(u_ref, w1_ref, w2_ref, o_ref, *, inv_n, bb):
    for b in range(bb):
        u = u_ref[b].astype(jnp.float32)
        z = (jnp.sum(u, axis=(1, 2), keepdims=True) * inv_n
             + jnp.max(u, axis=(1, 2), keepdims=True))
        zc = z[:, 0, :]
        h = jnp.dot(w1_ref[...], zc, preferred_element_type=jnp.float32)
        s = jnp.dot(w2_ref[...], h, preferred_element_type=jnp.float32)
        gate = jax.nn.sigmoid(s)
        o_ref[b] = jnp.broadcast_to(gate[:, :, None].astype(o_ref.dtype),
                                    o_ref.shape[1:])


def kernel(U, w_squeeze, w_excitation):
    B, C, H, W = U.shape
    bb = 2 if B % 2 == 0 else 1
    w1 = w_squeeze.reshape(C // 2, C).astype(jnp.float32)
    w2 = w_excitation.reshape(C, C // 2).astype(jnp.float32)

    return pl.pallas_call(
        functools.partial(_se_kernel, inv_n=1.0 / (H * W), bb=bb),
        out_shape=jax.ShapeDtypeStruct((B, C, H, W), U.dtype),
        grid=(B // bb,),
        in_specs=[
            pl.BlockSpec((bb, C, H, W), lambda b: (b, 0, 0, 0)),
            pl.BlockSpec((C // 2, C), lambda b: (0, 0)),
            pl.BlockSpec((C, C // 2), lambda b: (0, 0)),
        ],
        out_specs=pl.BlockSpec((bb, C, H, W), lambda b: (b, 0, 0, 0)),
        compiler_params=pltpu.CompilerParams(
            dimension_semantics=("parallel",)),
    )(U, w1, w2)


# speedup vs baseline: 6.2023x; 6.2023x over previous
import functools

import jax
import jax.numpy as jnp
from jax.experimental import pallas as pl
from jax.experimental.pallas import tpu as pltpu


def _se_kernel(u_ref, w1t_ref, w2t_ref, o_ref, *, inv_n, bb):
    for b in range(bb):
        u = u_ref[b].astype(jnp.float32)
        z = (jnp.sum(u, axis=(0, 1), keepdims=True) * inv_n
             + jnp.max(u, axis=(0, 1), keepdims=True))
        zr = z[0]
        h = jnp.dot(zr, w1t_ref[...], preferred_element_type=jnp.float32)
        s = jnp.dot(h, w2t_ref[...], preferred_element_type=jnp.float32)
        gate = jax.nn.sigmoid(s)
        o_ref[b] = jnp.broadcast_to(gate[None].astype(o_ref.dtype),
                                    o_ref.shape[1:])


def kernel(U, w_squeeze, w_excitation):
    B, C, H, W = U.shape
    bb = 2 if B % 2 == 0 else 1
    u_t = jnp.transpose(U, (0, 2, 3, 1))
    w1t = w_squeeze.reshape(C // 2, C).astype(jnp.float32).T
    w2t = w_excitation.reshape(C, C // 2).astype(jnp.float32).T

    out_t = pl.pallas_call(
        functools.partial(_se_kernel, inv_n=1.0 / (H * W), bb=bb),
        out_shape=jax.ShapeDtypeStruct((B, H, W, C), U.dtype),
        grid=(B // bb,),
        in_specs=[
            pl.BlockSpec((bb, H, W, C), lambda b: (b, 0, 0, 0)),
            pl.BlockSpec((C, C // 2), lambda b: (0, 0)),
            pl.BlockSpec((C // 2, C), lambda b: (0, 0)),
        ],
        out_specs=pl.BlockSpec((bb, H, W, C), lambda b: (b, 0, 0, 0)),
        compiler_params=pltpu.CompilerParams(
            dimension_semantics=("parallel",)),
    )(u_t, w1t, w2t)
    return jnp.transpose(out_t, (0, 3, 1, 2))


# speedup vs baseline: 6.6867x; 1.0781x over previous
import functools

import jax
import jax.numpy as jnp
from jax.experimental import pallas as pl
from jax.experimental.pallas import tpu as pltpu


def _se_kernel(u_ref, w1t_ref, w2t_ref, o_ref, *, inv_n, bb):
    for b in range(bb):
        u = u_ref[b].astype(jnp.float32)
        z = (jnp.sum(u, axis=(0, 1), keepdims=True) * inv_n
             + jnp.max(u, axis=(0, 1), keepdims=True))
        zr = z[0]
        h = jnp.dot(zr, w1t_ref[...], preferred_element_type=jnp.float32)
        s = jnp.dot(h, w2t_ref[...], preferred_element_type=jnp.float32)
        gate = jax.nn.sigmoid(s)
        o_ref[b] = jnp.broadcast_to(gate[None].astype(o_ref.dtype),
                                    o_ref.shape[1:])


def kernel(U, w_squeeze, w_excitation):
    B, C, H, W = U.shape
    bb = 4 if B % 4 == 0 else (2 if B % 2 == 0 else 1)
    u_t = jnp.transpose(U, (0, 2, 3, 1))
    w1t = w_squeeze.reshape(C // 2, C).astype(jnp.float32).T
    w2t = w_excitation.reshape(C, C // 2).astype(jnp.float32).T

    out_t = pl.pallas_call(
        functools.partial(_se_kernel, inv_n=1.0 / (H * W), bb=bb),
        out_shape=jax.ShapeDtypeStruct((B, H, W, C), U.dtype),
        grid=(B // bb,),
        in_specs=[
            pl.BlockSpec((bb, H, W, C), lambda b: (b, 0, 0, 0)),
            pl.BlockSpec((C, C // 2), lambda b: (0, 0)),
            pl.BlockSpec((C // 2, C), lambda b: (0, 0)),
        ],
        out_specs=pl.BlockSpec((bb, H, W, C), lambda b: (b, 0, 0, 0)),
        compiler_params=pltpu.CompilerParams(
            dimension_semantics=("parallel",)),
    )(u_t, w1t, w2t)
    return jnp.transpose(out_t, (0, 3, 1, 2))


# speedup vs baseline: 6.7062x; 1.0029x over previous
import functools

import jax
import jax.numpy as jnp
from jax.experimental import pallas as pl
from jax.experimental.pallas import tpu as pltpu


def _se_kernel(u_ref, w1t_ref, w2t_ref, o_ref, *, inv_n, bb):
    for b in range(bb):
        u = u_ref[b].astype(jnp.float32)
        z = (jnp.sum(u, axis=(0, 1), keepdims=True) * inv_n
             + jnp.max(u, axis=(0, 1), keepdims=True))
        zr = z[0]
        h = jnp.dot(zr, w1t_ref[...], preferred_element_type=jnp.float32)
        s = jnp.dot(h, w2t_ref[...], preferred_element_type=jnp.float32)
        gate = jax.nn.sigmoid(s)
        o_ref[b] = jnp.broadcast_to(gate[None].astype(o_ref.dtype),
                                    o_ref.shape[1:])


def kernel(U, w_squeeze, w_excitation):
    B, C, H, W = U.shape
    bb = 4 if B % 4 == 0 else (2 if B % 2 == 0 else 1)
    u_t = jnp.transpose(U, (0, 2, 3, 1))
    w1t = w_squeeze.reshape(C // 2, C).astype(jnp.float32).T
    w2t = jnp.transpose(w_excitation, (1, 0, 2, 3)).reshape(
        C // 2, C).astype(jnp.float32)

    out_t = pl.pallas_call(
        functools.partial(_se_kernel, inv_n=1.0 / (H * W), bb=bb),
        out_shape=jax.ShapeDtypeStruct((B, H, W, C), U.dtype),
        grid=(B // bb,),
        in_specs=[
            pl.BlockSpec((bb, H, W, C), lambda b: (b, 0, 0, 0)),
            pl.BlockSpec((C, C // 2), lambda b: (0, 0)),
            pl.BlockSpec((C // 2, C), lambda b: (0, 0)),
        ],
        out_specs=pl.BlockSpec((bb, H, W, C), lambda b: (b, 0, 0, 0)),
        compiler_params=pltpu.CompilerParams(
            dimension_semantics=("parallel",)),
    )(u_t, w1t, w2t)
    return jnp.transpose(out_t, (0, 3, 1, 2))
